# Initial kernel scaffold; baseline (speedup 1.0000x reference)
import jax
import jax.numpy as jnp
from jax.experimental import pallas as pl


def kernel(edge_index, h, e, params):
    raise NotImplementedError("write your pallas kernel here")


# speedup vs baseline: 7.3387x; 7.3387x over previous
import functools

import jax
import jax.numpy as jnp
from jax import lax
from jax.experimental import pallas as pl
from jax.experimental.pallas import tpu as pltpu
from jax.experimental.pallas import tpu_sc as plsc

_N = 10000
_E = 160000
_HID = 128
_F32 = jnp.float32

_NC = 2
_NS = 16
_NW = _NC * _NS
_CH = 128
_NCHUNK = 40
_EPT = _NCHUNK * _CH
_EPAD = _NW * _EPT
_NACC = 10240
_NCH_ACC = _NACC // _CH
_ZACC = _NACC // 16
_DUMMY = _N

_EBLK = 256
_EGRID = _EPAD // _EBLK
_EREAL = _E // _EBLK
_HBLK = 80
_HGRID = _N // _HBLK
_NBLK = 2000
_NGRID = _N // _NBLK


def _head_mat(rows, cols, div_axis):
    a = lax.broadcasted_iota(jnp.int32, (rows, cols), 0)
    b = lax.broadcasted_iota(jnp.int32, (rows, cols), 1)
    if div_axis == 1:
        b = b // 16
    else:
        a = a // 16
    return (a == b).astype(_F32)


def _expand_mats():
    bt = _head_mat(128, 8, 0)
    b8 = _head_mat(8, 128, 1)
    l8 = lax.broadcasted_iota(jnp.int32, (8, 128), 1)
    r8 = lax.broadcasted_iota(jnp.int32, (8, 128), 0)
    p1 = (l8 % 8 == r8).astype(_F32)
    l16 = lax.broadcasted_iota(jnp.int32, (16, 128), 1)
    r16 = lax.broadcasted_iota(jnp.int32, (16, 128), 0)
    p2 = (l16 // 8 == r16).astype(_F32)
    return bt, b8, p1, p2


def _row_spec(blk, width):
    return pl.BlockSpec((blk, width), lambda i: (i, 0))


def _full_spec(shape):
    return pl.BlockSpec(shape, lambda i: tuple(0 for _ in shape))


def _stats_update(st_ref, x, i, nreal_blocks):
    @pl.when(i == 0)
    def _():
        st_ref[...] = jnp.zeros_like(st_ref)

    @pl.when(i < nreal_blocks)
    def _():
        s = jnp.sum(x, axis=0, keepdims=True)
        s2 = jnp.sum(x * x, axis=0, keepdims=True)
        st_ref[0:1, :] += s
        st_ref[1:2, :] += s2


def _bn_apply(x, st, g, b, n):
    mean = st[0:1, :] * (1.0 / n)
    var = st[1:2, :] * (1.0 / n) - mean * mean
    return (x - mean) * lax.rsqrt(var + 1e-5) * g + b


def _linear(x, w, b, blk):
    rows, fin = x.shape
    fout = w.shape[1]

    def body(x_ref, w_ref, b_ref, o_ref):
        o_ref[...] = jnp.dot(x_ref[...], w_ref[...],
                             preferred_element_type=_F32) + b_ref[...]

    return pl.pallas_call(
        body,
        grid=(rows // blk,),
        in_specs=[_row_spec(blk, fin), _full_spec(w.shape), _full_spec((1, fout))],
        out_specs=_row_spec(blk, fout),
        out_shape=jax.ShapeDtypeStruct((rows, fout), _F32),
    )(x, w, b.reshape(1, -1))


def _qkv(h, wq, bq, wkv, bkv):
    def body(x_ref, wq_ref, bq_ref, wkv_ref, bkv_ref, q_ref, kv_ref):
        x = x_ref[...]
        q_ref[...] = jnp.dot(x, wq_ref[...], preferred_element_type=_F32) + bq_ref[...]
        kv_ref[...] = jnp.dot(x, wkv_ref[...], preferred_element_type=_F32) + bkv_ref[...]

    return pl.pallas_call(
        body,
        grid=(_NGRID,),
        in_specs=[_row_spec(_NBLK, 128), _full_spec((128, 128)), _full_spec((1, 128)),
                  _full_spec((128, 256)), _full_spec((1, 256))],
        out_specs=[_row_spec(_NBLK, 128), _row_spec(_NBLK, 256)],
        out_shape=[jax.ShapeDtypeStruct((_N, 128), _F32),
                   jax.ShapeDtypeStruct((_N, 256), _F32)],
    )(h, wq, bq.reshape(1, -1), wkv, bkv.reshape(1, -1))


def _score(qd, ksvs, pe, ohm, want_eatt):
    def body(qd_ref, ksvs_ref, pe_ref, ohm_ref, *outs):
        if want_eatt:
            eatt_ref, c_ref, cz_ref = outs
        else:
            c_ref, cz_ref = outs
        bt, b8, p1, p2 = _expand_mats()
        ks = ksvs_ref[:, :128]
        vs = ksvs_ref[:, 128:]
        score = qd_ref[...] * ks * 0.25 * pe_ref[...]
        if want_eatt:
            eatt_ref[...] = score
        srow = jnp.clip(jnp.dot(score, bt, preferred_element_type=_F32),
                        -5.0, 5.0)
        sc = jnp.exp(srow)
        scex = jnp.dot(sc, b8, preferred_element_type=_F32)
        c_ref[...] = scex * vs
        cz_ref[...] = (jnp.dot(sc, p1, preferred_element_type=_F32)
                       * jnp.dot(ohm_ref[...], p2, preferred_element_type=_F32))

    out_shape = [jax.ShapeDtypeStruct((_EPAD, 128), _F32),
                 jax.ShapeDtypeStruct((_EPAD, 128), _F32)]
    out_specs = [_row_spec(_EBLK, 128), _row_spec(_EBLK, 128)]
    if want_eatt:
        out_shape = [jax.ShapeDtypeStruct((_EPAD, 128), _F32)] + out_shape
        out_specs = [_row_spec(_EBLK, 128)] + out_specs
    return pl.pallas_call(
        body,
        grid=(_EGRID,),
        in_specs=[_row_spec(_EBLK, 128), _row_spec(_EBLK, 256),
                  _row_spec(_EBLK, 128), _row_spec(_EBLK, 16)],
        out_specs=out_specs,
        out_shape=out_shape,
    )(qd, ksvs, pe, ohm)


def _hatt(wvp, zp, hin, wo, bo):
    def body(wv_ref, z_ref, h_ref, wo_ref, bo_ref, t_ref, st_ref):
        i = pl.program_id(0)
        _, b8, _, _ = _expand_mats()
        wv = wv_ref[0] + wv_ref[1]
        z8 = z_ref[0] + z_ref[1]
        zex = jnp.dot(z8, b8, preferred_element_type=_F32)
        h_att = wv / (zex + 1e-6)
        t = h_ref[...] + jnp.dot(h_att, wo_ref[...],
                                 preferred_element_type=_F32) + bo_ref[...]
        t_ref[...] = t
        _stats_update(st_ref, t, i, _HGRID)

    return pl.pallas_call(
        body,
        grid=(_HGRID,),
        in_specs=[pl.BlockSpec((2, _HBLK, 128), lambda i: (0, i, 0)),
                  pl.BlockSpec((2, _HBLK, 8), lambda i: (0, i, 0)),
                  _row_spec(_HBLK, 128), _full_spec((128, 128)), _full_spec((1, 128))],
        out_specs=[_row_spec(_HBLK, 128), _full_spec((8, 128))],
        out_shape=[jax.ShapeDtypeStruct((_N, 128), _F32),
                   jax.ShapeDtypeStruct((8, 128), _F32)],
    )(wvp, zp, hin, wo, bo.reshape(1, -1))


def _resid_linear(xin, att, wo, bo, rows, blk, nreal_blocks):
    def body(x_ref, a_ref, wo_ref, bo_ref, t_ref, st_ref):
        i = pl.program_id(0)
        t = x_ref[...] + jnp.dot(a_ref[...], wo_ref[...],
                                 preferred_element_type=_F32) + bo_ref[...]
        t_ref[...] = t
        _stats_update(st_ref, t, i, nreal_blocks)

    return pl.pallas_call(
        body,
        grid=(rows // blk,),
        in_specs=[_row_spec(blk, 128), _row_spec(blk, 128),
                  _full_spec((128, 128)), _full_spec((1, 128))],
        out_specs=[_row_spec(blk, 128), _full_spec((8, 128))],
        out_shape=[jax.ShapeDtypeStruct((rows, 128), _F32),
                   jax.ShapeDtypeStruct((8, 128), _F32)],
    )(xin, att, wo, bo.reshape(1, -1))


def _bn_ffn(t, st, bnp, w1, b1, w2, b2, n, rows, blk, nreal_blocks):
    def body(t_ref, st_ref, g_ref, bb_ref, w1_ref, b1_ref, w2_ref, b2_ref,
             t2_ref, st2_ref):
        i = pl.program_id(0)
        x = _bn_apply(t_ref[...], st_ref[...], g_ref[...], bb_ref[...], n)
        u = jnp.maximum(jnp.dot(x, w1_ref[...], preferred_element_type=_F32)
                        + b1_ref[...], 0.0)
        t2 = x + jnp.dot(u, w2_ref[...], preferred_element_type=_F32) + b2_ref[...]
        t2_ref[...] = t2
        _stats_update(st2_ref, t2, i, nreal_blocks)

    return pl.pallas_call(
        body,
        grid=(rows // blk,),
        in_specs=[_row_spec(blk, 128), _full_spec((8, 128)),
                  _full_spec((1, 128)), _full_spec((1, 128)),
                  _full_spec((128, 256)), _full_spec((1, 256)),
                  _full_spec((256, 128)), _full_spec((1, 128))],
        out_specs=[_row_spec(blk, 128), _full_spec((8, 128))],
        out_shape=[jax.ShapeDtypeStruct((rows, 128), _F32),
                   jax.ShapeDtypeStruct((8, 128), _F32)],
    )(t, st, bnp["g"].reshape(1, -1), bnp["b"].reshape(1, -1),
      w1, b1.reshape(1, -1), w2, b2.reshape(1, -1))


def _bn_only(t, st, bnp, n, rows, blk):
    def body(t_ref, st_ref, g_ref, bb_ref, o_ref):
        o_ref[...] = _bn_apply(t_ref[...], st_ref[...], g_ref[...], bb_ref[...], n)

    return pl.pallas_call(
        body,
        grid=(rows // blk,),
        in_specs=[_row_spec(blk, 128), _full_spec((8, 128)),
                  _full_spec((1, 128)), _full_spec((1, 128))],
        out_specs=_row_spec(blk, 128),
        out_shape=jax.ShapeDtypeStruct((rows, 128), _F32),
    )(t, st, bnp["g"].reshape(1, -1), bnp["b"].reshape(1, -1))


def _mlp(hs, hd, m0w, m0b, m1w, m1b, m2w, m2b):
    def body(hs_ref, hd_ref, w0_ref, b0_ref, w1_ref, b1_ref, w2_ref, b2_ref,
             o_ref):
        x = jnp.concatenate([hs_ref[...], hd_ref[...]], axis=1)
        x = jnp.maximum(jnp.dot(x, w0_ref[...], preferred_element_type=_F32)
                        + b0_ref[...], 0.0)
        x = jnp.maximum(jnp.dot(x, w1_ref[...], preferred_element_type=_F32)
                        + b1_ref[...], 0.0)
        o_ref[...] = jnp.dot(x, w2_ref[...], preferred_element_type=_F32) + b2_ref[...]

    return pl.pallas_call(
        body,
        grid=(_EGRID,),
        in_specs=[_row_spec(_EBLK, 128), _row_spec(_EBLK, 128),
                  _full_spec((256, 128)), _full_spec((1, 128)),
                  _full_spec((128, 64)), _full_spec((1, 64)),
                  _full_spec((64, 2)), _full_spec((1, 2))],
        out_specs=_row_spec(_EBLK, 2),
        out_shape=jax.ShapeDtypeStruct((_EPAD, 2), _F32),
    )(hs, hd, m0w, m0b.reshape(1, -1), m1w, m1b.reshape(1, -1),
      m2w, m2b.reshape(1, -1))


def _sc_gather2(tab_a, idx_a, tab_b, idx_b):
    wa = tab_a.shape[1]
    wb = tab_b.shape[1]
    mesh = plsc.VectorSubcoreMesh(core_axis_name="c", subcore_axis_name="s", num_cores=_NC, num_subcores=_NS)

    @functools.partial(
        pl.kernel, mesh=mesh,
        out_type=[jax.ShapeDtypeStruct((_EPAD, wa), _F32),
                  jax.ShapeDtypeStruct((_EPAD, wb), _F32)],
        scratch_types=[pltpu.VMEM((_NCHUNK, _CH), jnp.int32),
                       pltpu.VMEM((_NCHUNK, _CH), jnp.int32),
                       pltpu.VMEM((_CH, wa), _F32),
                       pltpu.VMEM((_CH, wb), _F32),
                       pltpu.SemaphoreType.DMA],
    )
    def run(ta, ia, tb, ib, oa, ob, ia_v, ib_v, abuf, bbuf, sem):
        wid = lax.axis_index("s") * _NC + lax.axis_index("c")
        pltpu.sync_copy(ia.at[wid], ia_v)
        pltpu.sync_copy(ib.at[wid], ib_v)
        base = wid * _EPT

        def body(j, carry):
            off = base + j * _CH
            pltpu.async_copy(ta.at[ia_v.at[j]], abuf, sem).wait()
            pltpu.sync_copy(abuf, oa.at[pl.ds(off, _CH)])
            pltpu.async_copy(tb.at[ib_v.at[j]], bbuf, sem).wait()
            pltpu.sync_copy(bbuf, ob.at[pl.ds(off, _CH)])
            return carry

        lax.fori_loop(0, _NCHUNK, body, 0, unroll=False)

    return run(tab_a, idx_a, tab_b, idx_b)


def _sc_scatter(contrib, contribz, idx_s, idx_z, zeros128):
    mesh = plsc.VectorSubcoreMesh(core_axis_name="c", subcore_axis_name="s",
                                  num_cores=_NC, num_subcores=_NS)

    @functools.partial(
        pl.kernel, mesh=mesh,
        out_type=[jax.ShapeDtypeStruct((2, _NACC, 128), _F32),
                  jax.ShapeDtypeStruct((2, _ZACC, 128), _F32)],
        scratch_types=[pltpu.VMEM_SHARED((_NACC, 128), _F32),
                       pltpu.VMEM_SHARED((_ZACC, 128), _F32),
                       pltpu.VMEM((_NCHUNK, _CH), jnp.int32),
                       pltpu.VMEM((_NCHUNK, _CH), jnp.int32),
                       pltpu.VMEM((_CH, 128), _F32)],
    )
    def run(c_h, cz_h, idxs_h, idxz_h, z128, wv_out, z_out,
            accv, accz, idxs_v, idxz_v, cbuf):
        cid = lax.axis_index("c")
        sid = lax.axis_index("s")
        wid = sid * _NC + cid

        for k in range(_NCH_ACC // _NS):
            coff = (sid + k * _NS) * _CH
            pltpu.sync_copy(z128, accv.at[pl.ds(coff, _CH)])

        @pl.when(sid < _ZACC // _CH)
        def _():
            pltpu.sync_copy(z128, accz.at[pl.ds(sid * _CH, _CH)])

        plsc.subcore_barrier()

        pltpu.sync_copy(idxs_h.at[wid], idxs_v)
        pltpu.sync_copy(idxz_h.at[wid], idxz_v)
        base = wid * _EPT

        def body(j, carry):
            off = base + j * _CH
            pltpu.sync_copy(c_h.at[pl.ds(off, _CH)], cbuf)
            pltpu.sync_copy(cbuf, accv.at[idxs_v.at[j]], add=True)
            pltpu.sync_copy(cz_h.at[pl.ds(off, _CH)], cbuf)
            pltpu.sync_copy(cbuf, accz.at[idxz_v.at[j]], add=True)
            return carry

        lax.fori_loop(0, _NCHUNK, body, 0, unroll=False)
        plsc.subcore_barrier()

        for k in range(_NCH_ACC // _NS):
            coff = (sid + k * _NS) * _CH
            pltpu.sync_copy(accv.at[pl.ds(coff, _CH)], wv_out.at[cid, pl.ds(coff, _CH)])

        @pl.when(sid < _ZACC // _CH)
        def _():
            pltpu.sync_copy(accz.at[pl.ds(sid * _CH, _CH)],
                            z_out.at[cid, pl.ds(sid * _CH, _CH)])

    return run(contrib, contribz, idx_s, idx_z, zeros128)


def kernel(edge_index, h, e, params):
    src = edge_index[0]
    dst = edge_index[1]
    pad = _EPAD - _E
    srcp = jnp.concatenate([src, jnp.zeros((pad,), jnp.int32)]).reshape(
        _NW, _NCHUNK, _CH)
    dstp = jnp.concatenate([dst, jnp.zeros((pad,), jnp.int32)]).reshape(
        _NW, _NCHUNK, _CH)
    dstf = jnp.concatenate([dst, jnp.full((pad,), _DUMMY, jnp.int32)])
    dsts = dstf.reshape(_NW, _NCHUNK, _CH)
    dstz = (dstf // 16).reshape(_NW, _NCHUNK, _CH)
    ohm = jax.nn.one_hot(dstf % 16, 16, dtype=_F32)
    ep = jnp.concatenate([e, jnp.zeros((pad, e.shape[1]), e.dtype)])
    zeros128 = jnp.zeros((_CH, 128), _F32)

    h = _linear(h, params["emb_h"]["w"], params["emb_h"]["b"], _NBLK)
    e = _linear(ep, params["emb_e"]["w"], params["emb_e"]["b"], _EBLK)

    layers = params["layers"]
    nl = len(layers)
    for li, lp in enumerate(layers):
        last = li == nl - 1
        hin, ein = h, e
        q, kv = _qkv(h, lp["Q"]["w"], lp["Q"]["b"],
                     jnp.concatenate([lp["K"]["w"], lp["V"]["w"]], axis=1),
                     jnp.concatenate([lp["K"]["b"], lp["V"]["b"]]))
        pe = _linear(e, lp["Epj"]["w"], lp["Epj"]["b"], _EBLK)
        qd, ksvs = _sc_gather2(q, dstp, kv, srcp)
        if last:
            c, cz = _score(qd, ksvs, pe, ohm, want_eatt=False)
        else:
            e_att, c, cz = _score(qd, ksvs, pe, ohm, want_eatt=True)
        wvp, zp = _sc_scatter(c, cz, dsts, dstz, zeros128)
        zp = zp.reshape(2, _NACC, 8)
        t1h, st1h = _hatt(wvp, zp, hin, lp["Oh"]["w"], lp["Oh"]["b"])
        t2h, st2h = _bn_ffn(t1h, st1h, lp["bn1h"],
                            lp["ffn_h1"]["w"], lp["ffn_h1"]["b"],
                            lp["ffn_h2"]["w"], lp["ffn_h2"]["b"],
                            float(_N), _N, _HBLK, _HGRID)
        h = _bn_only(t2h, st2h, lp["bn2h"], float(_N), _N, _NBLK)
        if not last:
            t1e, st1e = _resid_linear(ein, e_att, lp["Oe"]["w"], lp["Oe"]["b"],
                                      _EPAD, _EBLK, _EREAL)
            t2e, st2e = _bn_ffn(t1e, st1e, lp["bn1e"],
                                lp["ffn_e1"]["w"], lp["ffn_e1"]["b"],
                                lp["ffn_e2"]["w"], lp["ffn_e2"]["b"],
                                float(_E), _EPAD, _EBLK, _EREAL)
            e = _bn_only(t2e, st2e, lp["bn2e"], float(_E), _EPAD, _EBLK)

    hs, hd = _sc_gather2(h, srcp, h, dstp)
    mlp = params["mlp"]
    x = _mlp(hs, hd, mlp[0]["w"], mlp[0]["b"], mlp[1]["w"], mlp[1]["b"],
             mlp[2]["w"], mlp[2]["b"])
    return x[:_E]


# speedup vs baseline: 7.7257x; 1.0527x over previous
import functools

import jax
import jax.numpy as jnp
from jax import lax
from jax.experimental import pallas as pl
from jax.experimental.pallas import tpu as pltpu
from jax.experimental.pallas import tpu_sc as plsc

_N = 10000
_E = 160000
_HID = 128
_F32 = jnp.float32

_NC = 2
_NS = 16
_NW = _NC * _NS
_CH = 128
_NCHUNK = 40
_EPT = _NCHUNK * _CH
_EPAD = _NW * _EPT
_NACC = 10240
_NCH_ACC = _NACC // _CH
_ZACC = _NACC // 16
_DUMMY = _N

_EBLK = 256
_EGRID = _EPAD // _EBLK
_EREAL = _E // _EBLK
_HBLK = 80
_HGRID = _N // _HBLK
_NBLK = 2000
_NGRID = _N // _NBLK


def _head_mat(rows, cols, div_axis):
    a = lax.broadcasted_iota(jnp.int32, (rows, cols), 0)
    b = lax.broadcasted_iota(jnp.int32, (rows, cols), 1)
    if div_axis == 1:
        b = b // 16
    else:
        a = a // 16
    return (a == b).astype(_F32)


def _expand_mats():
    bt = _head_mat(128, 8, 0)
    b8 = _head_mat(8, 128, 1)
    l8 = lax.broadcasted_iota(jnp.int32, (8, 128), 1)
    r8 = lax.broadcasted_iota(jnp.int32, (8, 128), 0)
    p1 = (l8 % 8 == r8).astype(_F32)
    l16 = lax.broadcasted_iota(jnp.int32, (16, 128), 1)
    r16 = lax.broadcasted_iota(jnp.int32, (16, 128), 0)
    p2 = (l16 // 8 == r16).astype(_F32)
    return bt, b8, p1, p2


def _row_spec(blk, width):
    return pl.BlockSpec((blk, width), lambda i: (i, 0))


def _full_spec(shape):
    return pl.BlockSpec(shape, lambda i: tuple(0 for _ in shape))


def _stats_update(st_ref, x, i, nreal_blocks):
    @pl.when(i == 0)
    def _():
        st_ref[...] = jnp.zeros_like(st_ref)

    @pl.when(i < nreal_blocks)
    def _():
        s = jnp.sum(x, axis=0, keepdims=True)
        s2 = jnp.sum(x * x, axis=0, keepdims=True)
        st_ref[0:1, :] += s
        st_ref[1:2, :] += s2


def _bn_apply(x, st, g, b, n):
    mean = st[0:1, :] * (1.0 / n)
    var = st[1:2, :] * (1.0 / n) - mean * mean
    return (x - mean) * lax.rsqrt(var + 1e-5) * g + b


def _linear(x, w, b, blk):
    rows, fin = x.shape
    fout = w.shape[1]

    def body(x_ref, w_ref, b_ref, o_ref):
        o_ref[...] = jnp.dot(x_ref[...], w_ref[...],
                             preferred_element_type=_F32) + b_ref[...]

    return pl.pallas_call(
        body,
        grid=(rows // blk,),
        in_specs=[_row_spec(blk, fin), _full_spec(w.shape), _full_spec((1, fout))],
        out_specs=_row_spec(blk, fout),
        out_shape=jax.ShapeDtypeStruct((rows, fout), _F32),
    )(x, w, b.reshape(1, -1))


def _qkv(h, wq, bq, wkv, bkv):
    def body(x_ref, wq_ref, bq_ref, wkv_ref, bkv_ref, q_ref, kv_ref):
        x = x_ref[...]
        q_ref[...] = jnp.dot(x, wq_ref[...], preferred_element_type=_F32) + bq_ref[...]
        kv_ref[...] = jnp.dot(x, wkv_ref[...], preferred_element_type=_F32) + bkv_ref[...]

    return pl.pallas_call(
        body,
        grid=(_NGRID,),
        in_specs=[_row_spec(_NBLK, 128), _full_spec((128, 128)), _full_spec((1, 128)),
                  _full_spec((128, 256)), _full_spec((1, 256))],
        out_specs=[_row_spec(_NBLK, 128), _row_spec(_NBLK, 256)],
        out_shape=[jax.ShapeDtypeStruct((_N, 128), _F32),
                   jax.ShapeDtypeStruct((_N, 256), _F32)],
    )(h, wq, bq.reshape(1, -1), wkv, bkv.reshape(1, -1))


def _score(qd, ksvs, pe, ohm, want_eatt):
    def body(qd_ref, ksvs_ref, pe_ref, ohm_ref, *outs):
        if want_eatt:
            eatt_ref, c_ref, cz_ref = outs
        else:
            c_ref, cz_ref = outs
        bt, b8, p1, p2 = _expand_mats()
        ks = ksvs_ref[:, :128]
        vs = ksvs_ref[:, 128:]
        score = qd_ref[...] * ks * 0.25 * pe_ref[...]
        if want_eatt:
            eatt_ref[...] = score
        srow = jnp.clip(jnp.dot(score, bt, preferred_element_type=_F32),
                        -5.0, 5.0)
        sc = jnp.exp(srow)
        scex = jnp.dot(sc, b8, preferred_element_type=_F32)
        c_ref[...] = scex * vs
        cz_ref[...] = (jnp.dot(sc, p1, preferred_element_type=_F32)
                       * jnp.dot(ohm_ref[...], p2, preferred_element_type=_F32))

    out_shape = [jax.ShapeDtypeStruct((_EPAD, 128), _F32),
                 jax.ShapeDtypeStruct((_EPAD, 128), _F32)]
    out_specs = [_row_spec(_EBLK, 128), _row_spec(_EBLK, 128)]
    if want_eatt:
        out_shape = [jax.ShapeDtypeStruct((_EPAD, 128), _F32)] + out_shape
        out_specs = [_row_spec(_EBLK, 128)] + out_specs
    return pl.pallas_call(
        body,
        grid=(_EGRID,),
        in_specs=[_row_spec(_EBLK, 128), _row_spec(_EBLK, 256),
                  _row_spec(_EBLK, 128), _row_spec(_EBLK, 16)],
        out_specs=out_specs,
        out_shape=out_shape,
    )(qd, ksvs, pe, ohm)


def _hatt(wvp, zp, hin, wo, bo):
    def body(wv_ref, z_ref, h_ref, wo_ref, bo_ref, t_ref, st_ref):
        i = pl.program_id(0)
        _, b8, _, _ = _expand_mats()
        wv = wv_ref[0] + wv_ref[1]
        z8 = z_ref[0] + z_ref[1]
        zex = jnp.dot(z8, b8, preferred_element_type=_F32)
        h_att = wv / (zex + 1e-6)
        t = h_ref[...] + jnp.dot(h_att, wo_ref[...],
                                 preferred_element_type=_F32) + bo_ref[...]
        t_ref[...] = t
        _stats_update(st_ref, t, i, _HGRID)

    return pl.pallas_call(
        body,
        grid=(_HGRID,),
        in_specs=[pl.BlockSpec((2, _HBLK, 128), lambda i: (0, i, 0)),
                  pl.BlockSpec((2, _HBLK, 8), lambda i: (0, i, 0)),
                  _row_spec(_HBLK, 128), _full_spec((128, 128)), _full_spec((1, 128))],
        out_specs=[_row_spec(_HBLK, 128), _full_spec((8, 128))],
        out_shape=[jax.ShapeDtypeStruct((_N, 128), _F32),
                   jax.ShapeDtypeStruct((8, 128), _F32)],
    )(wvp, zp, hin, wo, bo.reshape(1, -1))


def _resid_linear(xin, att, wo, bo, rows, blk, nreal_blocks):
    def body(x_ref, a_ref, wo_ref, bo_ref, t_ref, st_ref):
        i = pl.program_id(0)
        t = x_ref[...] + jnp.dot(a_ref[...], wo_ref[...],
                                 preferred_element_type=_F32) + bo_ref[...]
        t_ref[...] = t
        _stats_update(st_ref, t, i, nreal_blocks)

    return pl.pallas_call(
        body,
        grid=(rows // blk,),
        in_specs=[_row_spec(blk, 128), _row_spec(blk, 128),
                  _full_spec((128, 128)), _full_spec((1, 128))],
        out_specs=[_row_spec(blk, 128), _full_spec((8, 128))],
        out_shape=[jax.ShapeDtypeStruct((rows, 128), _F32),
                   jax.ShapeDtypeStruct((8, 128), _F32)],
    )(xin, att, wo, bo.reshape(1, -1))


def _bn_ffn(t, st, bnp, w1, b1, w2, b2, n, rows, blk, nreal_blocks):
    def body(t_ref, st_ref, g_ref, bb_ref, w1_ref, b1_ref, w2_ref, b2_ref,
             t2_ref, st2_ref):
        i = pl.program_id(0)
        x = _bn_apply(t_ref[...], st_ref[...], g_ref[...], bb_ref[...], n)
        u = jnp.maximum(jnp.dot(x, w1_ref[...], preferred_element_type=_F32)
                        + b1_ref[...], 0.0)
        t2 = x + jnp.dot(u, w2_ref[...], preferred_element_type=_F32) + b2_ref[...]
        t2_ref[...] = t2
        _stats_update(st2_ref, t2, i, nreal_blocks)

    return pl.pallas_call(
        body,
        grid=(rows // blk,),
        in_specs=[_row_spec(blk, 128), _full_spec((8, 128)),
                  _full_spec((1, 128)), _full_spec((1, 128)),
                  _full_spec((128, 256)), _full_spec((1, 256)),
                  _full_spec((256, 128)), _full_spec((1, 128))],
        out_specs=[_row_spec(blk, 128), _full_spec((8, 128))],
        out_shape=[jax.ShapeDtypeStruct((rows, 128), _F32),
                   jax.ShapeDtypeStruct((8, 128), _F32)],
    )(t, st, bnp["g"].reshape(1, -1), bnp["b"].reshape(1, -1),
      w1, b1.reshape(1, -1), w2, b2.reshape(1, -1))


def _bn_only(t, st, bnp, n, rows, blk):
    def body(t_ref, st_ref, g_ref, bb_ref, o_ref):
        o_ref[...] = _bn_apply(t_ref[...], st_ref[...], g_ref[...], bb_ref[...], n)

    return pl.pallas_call(
        body,
        grid=(rows // blk,),
        in_specs=[_row_spec(blk, 128), _full_spec((8, 128)),
                  _full_spec((1, 128)), _full_spec((1, 128))],
        out_specs=_row_spec(blk, 128),
        out_shape=jax.ShapeDtypeStruct((rows, 128), _F32),
    )(t, st, bnp["g"].reshape(1, -1), bnp["b"].reshape(1, -1))


def _mlp(hs, hd, m0w, m0b, m1w, m1b, m2w, m2b):
    def body(hs_ref, hd_ref, w0_ref, b0_ref, w1_ref, b1_ref, w2_ref, b2_ref,
             o_ref):
        x = jnp.concatenate([hs_ref[...], hd_ref[...]], axis=1)
        x = jnp.maximum(jnp.dot(x, w0_ref[...], preferred_element_type=_F32)
                        + b0_ref[...], 0.0)
        x = jnp.maximum(jnp.dot(x, w1_ref[...], preferred_element_type=_F32)
                        + b1_ref[...], 0.0)
        o_ref[...] = jnp.dot(x, w2_ref[...], preferred_element_type=_F32) + b2_ref[...]

    return pl.pallas_call(
        body,
        grid=(_EGRID,),
        in_specs=[_row_spec(_EBLK, 128), _row_spec(_EBLK, 128),
                  _full_spec((256, 128)), _full_spec((1, 128)),
                  _full_spec((128, 64)), _full_spec((1, 64)),
                  _full_spec((64, 2)), _full_spec((1, 2))],
        out_specs=_row_spec(_EBLK, 2),
        out_shape=jax.ShapeDtypeStruct((_EPAD, 2), _F32),
    )(hs, hd, m0w, m0b.reshape(1, -1), m1w, m1b.reshape(1, -1),
      m2w, m2b.reshape(1, -1))


def _sc_gather2(tab_a, idx_a, tab_b, idx_b):
    wa = tab_a.shape[1]
    wb = tab_b.shape[1]
    mesh = plsc.VectorSubcoreMesh(core_axis_name="c", subcore_axis_name="s", num_cores=_NC, num_subcores=_NS)

    @functools.partial(
        pl.kernel, mesh=mesh,
        out_type=[jax.ShapeDtypeStruct((_EPAD, wa), _F32),
                  jax.ShapeDtypeStruct((_EPAD, wb), _F32)],
        scratch_types=[pltpu.VMEM((_NCHUNK, _CH), jnp.int32),
                       pltpu.VMEM((_NCHUNK, _CH), jnp.int32),
                       pltpu.VMEM((_CH, wa), _F32),
                       pltpu.VMEM((_CH, wa), _F32),
                       pltpu.VMEM((_CH, wb), _F32),
                       pltpu.VMEM((_CH, wb), _F32),
                       pltpu.SemaphoreType.DMA, pltpu.SemaphoreType.DMA,
                       pltpu.SemaphoreType.DMA, pltpu.SemaphoreType.DMA,
                       pltpu.SemaphoreType.DMA, pltpu.SemaphoreType.DMA,
                       pltpu.SemaphoreType.DMA, pltpu.SemaphoreType.DMA],
    )
    def run(ta, ia, tb, ib, oa, ob, ia_v, ib_v, a0, a1, b0, b1,
            gsa0, gsa1, gsb0, gsb1, wsa0, wsa1, wsb0, wsb1):
        wid = lax.axis_index("s") * _NC + lax.axis_index("c")
        pltpu.sync_copy(ia.at[wid], ia_v)
        pltpu.sync_copy(ib.at[wid], ib_v)
        base = wid * _EPT
        abufs, bbufs = (a0, a1), (b0, b1)
        gsa, gsb = (gsa0, gsa1), (gsb0, gsb1)
        wsa, wsb = (wsa0, wsa1), (wsb0, wsb1)

        def body(i, carry):
            descs = []
            for p in (0, 1):
                j = 2 * i + p

                @pl.when(i > 0)
                def _(p=p):
                    pltpu.make_async_copy(abufs[p], oa.at[pl.ds(0, _CH)],
                                          wsa[p]).wait()
                    pltpu.make_async_copy(bbufs[p], ob.at[pl.ds(0, _CH)],
                                          wsb[p]).wait()

                da = pltpu.async_copy(ta.at[ia_v.at[j]], abufs[p], gsa[p])
                db = pltpu.async_copy(tb.at[ib_v.at[j]], bbufs[p], gsb[p])
                descs.append((da, db))
            for p in (0, 1):
                off = base + (2 * i + p) * _CH
                da, db = descs[p]
                da.wait()
                pltpu.async_copy(abufs[p], oa.at[pl.ds(off, _CH)], wsa[p])
                db.wait()
                pltpu.async_copy(bbufs[p], ob.at[pl.ds(off, _CH)], wsb[p])
            return carry

        lax.fori_loop(0, _NCHUNK // 2, body, 0, unroll=False)
        for p in (0, 1):
            pltpu.make_async_copy(abufs[p], oa.at[pl.ds(0, _CH)], wsa[p]).wait()
            pltpu.make_async_copy(bbufs[p], ob.at[pl.ds(0, _CH)], wsb[p]).wait()

    return run(tab_a, idx_a, tab_b, idx_b)


def _sc_scatter(contrib, contribz, idx_s, idx_z, zeros128):
    mesh = plsc.VectorSubcoreMesh(core_axis_name="c", subcore_axis_name="s",
                                  num_cores=_NC, num_subcores=_NS)

    @functools.partial(
        pl.kernel, mesh=mesh,
        out_type=[jax.ShapeDtypeStruct((2, _NACC, 128), _F32),
                  jax.ShapeDtypeStruct((2, _ZACC, 128), _F32)],
        scratch_types=[pltpu.VMEM_SHARED((_NACC, 128), _F32),
                       pltpu.VMEM_SHARED((_ZACC, 128), _F32),
                       pltpu.VMEM((_NCHUNK, _CH), jnp.int32),
                       pltpu.VMEM((_NCHUNK, _CH), jnp.int32),
                       pltpu.VMEM((_CH, 128), _F32),
                       pltpu.VMEM((_CH, 128), _F32),
                       pltpu.SemaphoreType.DMA, pltpu.SemaphoreType.DMA],
    )
    def run(c_h, cz_h, idxs_h, idxz_h, z128, wv_out, z_out,
            accv, accz, idxs_v, idxz_v, cb, zb, lsc, lsz):
        cid = lax.axis_index("c")
        sid = lax.axis_index("s")
        wid = sid * _NC + cid

        for k in range(_NCH_ACC // _NS):
            coff = (sid + k * _NS) * _CH
            pltpu.sync_copy(z128, accv.at[pl.ds(coff, _CH)])

        @pl.when(sid < _ZACC // _CH)
        def _():
            pltpu.sync_copy(z128, accz.at[pl.ds(sid * _CH, _CH)])

        plsc.subcore_barrier()

        pltpu.sync_copy(idxs_h.at[wid], idxs_v)
        pltpu.sync_copy(idxz_h.at[wid], idxz_v)
        base = wid * _EPT
        pltpu.async_copy(c_h.at[pl.ds(base, _CH)], cb, lsc)

        def body(j, carry):
            off = base + j * _CH
            dz = pltpu.async_copy(cz_h.at[pl.ds(off, _CH)], zb, lsz)
            pltpu.make_async_copy(c_h.at[pl.ds(0, _CH)], cb, lsc).wait()
            pltpu.sync_copy(cb, accv.at[idxs_v.at[j]], add=True)

            @pl.when(j < _NCHUNK - 1)
            def _():
                pltpu.async_copy(c_h.at[pl.ds(off + _CH, _CH)], cb, lsc)

            dz.wait()
            pltpu.sync_copy(zb, accz.at[idxz_v.at[j]], add=True)
            return carry

        lax.fori_loop(0, _NCHUNK, body, 0, unroll=False)
        plsc.subcore_barrier()

        for k in range(_NCH_ACC // _NS):
            coff = (sid + k * _NS) * _CH
            pltpu.sync_copy(accv.at[pl.ds(coff, _CH)], wv_out.at[cid, pl.ds(coff, _CH)])

        @pl.when(sid < _ZACC // _CH)
        def _():
            pltpu.sync_copy(accz.at[pl.ds(sid * _CH, _CH)],
                            z_out.at[cid, pl.ds(sid * _CH, _CH)])

    return run(contrib, contribz, idx_s, idx_z, zeros128)


def kernel(edge_index, h, e, params):
    src = edge_index[0]
    dst = edge_index[1]
    pad = _EPAD - _E
    srcp = jnp.concatenate([src, jnp.zeros((pad,), jnp.int32)]).reshape(
        _NW, _NCHUNK, _CH)
    dstp = jnp.concatenate([dst, jnp.zeros((pad,), jnp.int32)]).reshape(
        _NW, _NCHUNK, _CH)
    dstf = jnp.concatenate([dst, jnp.full((pad,), _DUMMY, jnp.int32)])
    dsts = dstf.reshape(_NW, _NCHUNK, _CH)
    dstz = (dstf // 16).reshape(_NW, _NCHUNK, _CH)
    ohm = jax.nn.one_hot(dstf % 16, 16, dtype=_F32)
    ep = jnp.concatenate([e, jnp.zeros((pad, e.shape[1]), e.dtype)])
    zeros128 = jnp.zeros((_CH, 128), _F32)

    h = _linear(h, params["emb_h"]["w"], params["emb_h"]["b"], _NBLK)
    e = _linear(ep, params["emb_e"]["w"], params["emb_e"]["b"], _EBLK)

    layers = params["layers"]
    nl = len(layers)
    for li, lp in enumerate(layers):
        last = li == nl - 1
        hin, ein = h, e
        q, kv = _qkv(h, lp["Q"]["w"], lp["Q"]["b"],
                     jnp.concatenate([lp["K"]["w"], lp["V"]["w"]], axis=1),
                     jnp.concatenate([lp["K"]["b"], lp["V"]["b"]]))
        pe = _linear(e, lp["Epj"]["w"], lp["Epj"]["b"], _EBLK)
        qd, ksvs = _sc_gather2(q, dstp, kv, srcp)
        if last:
            c, cz = _score(qd, ksvs, pe, ohm, want_eatt=False)
        else:
            e_att, c, cz = _score(qd, ksvs, pe, ohm, want_eatt=True)
        wvp, zp = _sc_scatter(c, cz, dsts, dstz, zeros128)
        zp = zp.reshape(2, _NACC, 8)
        t1h, st1h = _hatt(wvp, zp, hin, lp["Oh"]["w"], lp["Oh"]["b"])
        t2h, st2h = _bn_ffn(t1h, st1h, lp["bn1h"],
                            lp["ffn_h1"]["w"], lp["ffn_h1"]["b"],
                            lp["ffn_h2"]["w"], lp["ffn_h2"]["b"],
                            float(_N), _N, _HBLK, _HGRID)
        h = _bn_only(t2h, st2h, lp["bn2h"], float(_N), _N, _NBLK)
        if not last:
            t1e, st1e = _resid_linear(ein, e_att, lp["Oe"]["w"], lp["Oe"]["b"],
                                      _EPAD, _EBLK, _EREAL)
            t2e, st2e = _bn_ffn(t1e, st1e, lp["bn1e"],
                                lp["ffn_e1"]["w"], lp["ffn_e1"]["b"],
                                lp["ffn_e2"]["w"], lp["ffn_e2"]["b"],
                                float(_E), _EPAD, _EBLK, _EREAL)
            e = _bn_only(t2e, st2e, lp["bn2e"], float(_E), _EPAD, _EBLK)

    hs, hd = _sc_gather2(h, srcp, h, dstp)
    mlp = params["mlp"]
    x = _mlp(hs, hd, mlp[0]["w"], mlp[0]["b"], mlp[1]["w"], mlp[1]["b"],
             mlp[2]["w"], mlp[2]["b"])
    return x[:_E]


# speedup vs baseline: 16.0278x; 2.0746x over previous
import functools

import jax
import jax.numpy as jnp
from jax import lax
from jax.experimental import pallas as pl
from jax.experimental.pallas import tpu as pltpu
from jax.experimental.pallas import tpu_sc as plsc

_N = 10000
_E = 160000
_HID = 128
_F32 = jnp.float32

_NC = 2
_NS = 16
_NW = _NC * _NS
_CH = 128
_NCHUNK = 40
_EPT = _NCHUNK * _CH
_EPAD = _NW * _EPT
_NACC = 10240
_NCH_ACC = _NACC // _CH
_ZACC = _NACC // 16
_DUMMY = _N

_EBLK = 1280
_EGRID = _EPAD // _EBLK
_EREAL = _E // _EBLK
_HBLK = 400
_HGRID = _N // _HBLK
_NBLK = 2000
_NGRID = _N // _NBLK


def _head_mat(rows, cols, div_axis):
    a = lax.broadcasted_iota(jnp.int32, (rows, cols), 0)
    b = lax.broadcasted_iota(jnp.int32, (rows, cols), 1)
    if div_axis == 1:
        b = b // 16
    else:
        a = a // 16
    return (a == b).astype(_F32)


def _expand_mats():
    bt = _head_mat(128, 8, 0)
    b8 = _head_mat(8, 128, 1)
    l8 = lax.broadcasted_iota(jnp.int32, (8, 128), 1)
    r8 = lax.broadcasted_iota(jnp.int32, (8, 128), 0)
    p1 = (l8 % 8 == r8).astype(_F32)
    l16 = lax.broadcasted_iota(jnp.int32, (16, 128), 1)
    r16 = lax.broadcasted_iota(jnp.int32, (16, 128), 0)
    p2 = (l16 // 8 == r16).astype(_F32)
    return bt, b8, p1, p2


def _row_spec(blk, width):
    return pl.BlockSpec((blk, width), lambda i: (i, 0))


def _full_spec(shape):
    return pl.BlockSpec(shape, lambda i: tuple(0 for _ in shape))


def _stats_update(st_ref, x, i, nreal_blocks):
    @pl.when(i == 0)
    def _():
        st_ref[...] = jnp.zeros_like(st_ref)

    @pl.when(i < nreal_blocks)
    def _():
        s = jnp.sum(x, axis=0, keepdims=True)
        s2 = jnp.sum(x * x, axis=0, keepdims=True)
        st_ref[0:1, :] += s
        st_ref[1:2, :] += s2


def _bn_apply(x, st, g, b, n):
    mean = st[0:1, :] * (1.0 / n)
    var = st[1:2, :] * (1.0 / n) - mean * mean
    return (x - mean) * lax.rsqrt(var + 1e-5) * g + b


def _linear(x, w, b, blk):
    rows, fin = x.shape
    fout = w.shape[1]

    def body(x_ref, w_ref, b_ref, o_ref):
        o_ref[...] = jnp.dot(x_ref[...], w_ref[...],
                             preferred_element_type=_F32) + b_ref[...]

    return pl.pallas_call(
        body,
        grid=(rows // blk,),
        in_specs=[_row_spec(blk, fin), _full_spec(w.shape), _full_spec((1, fout))],
        out_specs=_row_spec(blk, fout),
        out_shape=jax.ShapeDtypeStruct((rows, fout), _F32),
    )(x, w, b.reshape(1, -1))


def _qkv(h, wq, bq, wkv, bkv):
    def body(x_ref, wq_ref, bq_ref, wkv_ref, bkv_ref, q_ref, kv_ref):
        x = x_ref[...]
        q_ref[...] = jnp.dot(x, wq_ref[...], preferred_element_type=_F32) + bq_ref[...]
        kv_ref[...] = jnp.dot(x, wkv_ref[...], preferred_element_type=_F32) + bkv_ref[...]

    return pl.pallas_call(
        body,
        grid=(_NGRID,),
        in_specs=[_row_spec(_NBLK, 128), _full_spec((128, 128)), _full_spec((1, 128)),
                  _full_spec((128, 256)), _full_spec((1, 256))],
        out_specs=[_row_spec(_NBLK, 128), _row_spec(_NBLK, 256)],
        out_shape=[jax.ShapeDtypeStruct((_N, 128), _F32),
                   jax.ShapeDtypeStruct((_N, 256), _F32)],
    )(h, wq, bq.reshape(1, -1), wkv, bkv.reshape(1, -1))


def _score(qd, ksvs, pe, ohm, want_eatt):
    def body(qd_ref, ksvs_ref, pe_ref, ohm_ref, *outs):
        if want_eatt:
            eatt_ref, c_ref, cz_ref = outs
        else:
            c_ref, cz_ref = outs
        bt, b8, p1, p2 = _expand_mats()
        ks = ksvs_ref[:, :128]
        vs = ksvs_ref[:, 128:]
        score = qd_ref[...] * ks * 0.25 * pe_ref[...]
        if want_eatt:
            eatt_ref[...] = score
        srow = jnp.clip(jnp.dot(score, bt, preferred_element_type=_F32),
                        -5.0, 5.0)
        sc = jnp.exp(srow)
        scex = jnp.dot(sc, b8, preferred_element_type=_F32)
        c_ref[...] = scex * vs
        cz_ref[...] = (jnp.dot(sc, p1, preferred_element_type=_F32)
                       * jnp.dot(ohm_ref[...], p2, preferred_element_type=_F32))

    out_shape = [jax.ShapeDtypeStruct((_EPAD, 128), _F32),
                 jax.ShapeDtypeStruct((_EPAD, 128), _F32)]
    out_specs = [_row_spec(_EBLK, 128), _row_spec(_EBLK, 128)]
    if want_eatt:
        out_shape = [jax.ShapeDtypeStruct((_EPAD, 128), _F32)] + out_shape
        out_specs = [_row_spec(_EBLK, 128)] + out_specs
    return pl.pallas_call(
        body,
        grid=(_EGRID,),
        in_specs=[_row_spec(_EBLK, 128), _row_spec(_EBLK, 256),
                  _row_spec(_EBLK, 128), _row_spec(_EBLK, 16)],
        out_specs=out_specs,
        out_shape=out_shape,
    )(qd, ksvs, pe, ohm)


def _hatt(wvp, zp, hin, wo, bo):
    def body(wv_ref, z_ref, h_ref, wo_ref, bo_ref, t_ref, st_ref):
        i = pl.program_id(0)
        _, b8, _, _ = _expand_mats()
        wv = wv_ref[0] + wv_ref[1]
        z8 = z_ref[0] + z_ref[1]
        zex = jnp.dot(z8, b8, preferred_element_type=_F32)
        h_att = wv / (zex + 1e-6)
        t = h_ref[...] + jnp.dot(h_att, wo_ref[...],
                                 preferred_element_type=_F32) + bo_ref[...]
        t_ref[...] = t
        _stats_update(st_ref, t, i, _HGRID)

    return pl.pallas_call(
        body,
        grid=(_HGRID,),
        in_specs=[pl.BlockSpec((2, _HBLK, 128), lambda i: (0, i, 0)),
                  pl.BlockSpec((2, _HBLK, 8), lambda i: (0, i, 0)),
                  _row_spec(_HBLK, 128), _full_spec((128, 128)), _full_spec((1, 128))],
        out_specs=[_row_spec(_HBLK, 128), _full_spec((8, 128))],
        out_shape=[jax.ShapeDtypeStruct((_N, 128), _F32),
                   jax.ShapeDtypeStruct((8, 128), _F32)],
    )(wvp, zp, hin, wo, bo.reshape(1, -1))


def _resid_linear(xin, att, wo, bo, rows, blk, nreal_blocks):
    def body(x_ref, a_ref, wo_ref, bo_ref, t_ref, st_ref):
        i = pl.program_id(0)
        t = x_ref[...] + jnp.dot(a_ref[...], wo_ref[...],
                                 preferred_element_type=_F32) + bo_ref[...]
        t_ref[...] = t
        _stats_update(st_ref, t, i, nreal_blocks)

    return pl.pallas_call(
        body,
        grid=(rows // blk,),
        in_specs=[_row_spec(blk, 128), _row_spec(blk, 128),
                  _full_spec((128, 128)), _full_spec((1, 128))],
        out_specs=[_row_spec(blk, 128), _full_spec((8, 128))],
        out_shape=[jax.ShapeDtypeStruct((rows, 128), _F32),
                   jax.ShapeDtypeStruct((8, 128), _F32)],
    )(xin, att, wo, bo.reshape(1, -1))


def _bn_ffn(t, st, bnp, w1, b1, w2, b2, n, rows, blk, nreal_blocks):
    def body(t_ref, st_ref, g_ref, bb_ref, w1_ref, b1_ref, w2_ref, b2_ref,
             t2_ref, st2_ref):
        i = pl.program_id(0)
        x = _bn_apply(t_ref[...], st_ref[...], g_ref[...], bb_ref[...], n)
        u = jnp.maximum(jnp.dot(x, w1_ref[...], preferred_element_type=_F32)
                        + b1_ref[...], 0.0)
        t2 = x + jnp.dot(u, w2_ref[...], preferred_element_type=_F32) + b2_ref[...]
        t2_ref[...] = t2
        _stats_update(st2_ref, t2, i, nreal_blocks)

    return pl.pallas_call(
        body,
        grid=(rows // blk,),
        in_specs=[_row_spec(blk, 128), _full_spec((8, 128)),
                  _full_spec((1, 128)), _full_spec((1, 128)),
                  _full_spec((128, 256)), _full_spec((1, 256)),
                  _full_spec((256, 128)), _full_spec((1, 128))],
        out_specs=[_row_spec(blk, 128), _full_spec((8, 128))],
        out_shape=[jax.ShapeDtypeStruct((rows, 128), _F32),
                   jax.ShapeDtypeStruct((8, 128), _F32)],
    )(t, st, bnp["g"].reshape(1, -1), bnp["b"].reshape(1, -1),
      w1, b1.reshape(1, -1), w2, b2.reshape(1, -1))


def _bn_only(t, st, bnp, n, rows, blk):
    def body(t_ref, st_ref, g_ref, bb_ref, o_ref):
        o_ref[...] = _bn_apply(t_ref[...], st_ref[...], g_ref[...], bb_ref[...], n)

    return pl.pallas_call(
        body,
        grid=(rows // blk,),
        in_specs=[_row_spec(blk, 128), _full_spec((8, 128)),
                  _full_spec((1, 128)), _full_spec((1, 128))],
        out_specs=_row_spec(blk, 128),
        out_shape=jax.ShapeDtypeStruct((rows, 128), _F32),
    )(t, st, bnp["g"].reshape(1, -1), bnp["b"].reshape(1, -1))


def _bn_qkv(t, st, bnp, wq, bq, wkv, bkv):
    def body(t_ref, st_ref, g_ref, bb_ref, wq_ref, bq_ref, wkv_ref, bkv_ref,
             h_ref, q_ref, kv_ref):
        x = _bn_apply(t_ref[...], st_ref[...], g_ref[...], bb_ref[...],
                      float(_N))
        h_ref[...] = x
        q_ref[...] = jnp.dot(x, wq_ref[...], preferred_element_type=_F32) + bq_ref[...]
        kv_ref[...] = jnp.dot(x, wkv_ref[...], preferred_element_type=_F32) + bkv_ref[...]

    return pl.pallas_call(
        body,
        grid=(_NGRID,),
        in_specs=[_row_spec(_NBLK, 128), _full_spec((8, 128)),
                  _full_spec((1, 128)), _full_spec((1, 128)),
                  _full_spec((128, 128)), _full_spec((1, 128)),
                  _full_spec((128, 256)), _full_spec((1, 256))],
        out_specs=[_row_spec(_NBLK, 128), _row_spec(_NBLK, 128),
                   _row_spec(_NBLK, 256)],
        out_shape=[jax.ShapeDtypeStruct((_N, 128), _F32),
                   jax.ShapeDtypeStruct((_N, 128), _F32),
                   jax.ShapeDtypeStruct((_N, 256), _F32)],
    )(t, st, bnp["g"].reshape(1, -1), bnp["b"].reshape(1, -1),
      wq, bq.reshape(1, -1), wkv, bkv.reshape(1, -1))


def _bn_linear2(t, st, bnp, w, b, n):
    def body(t_ref, st_ref, g_ref, bb_ref, w_ref, b_ref, x_ref, y_ref):
        x = _bn_apply(t_ref[...], st_ref[...], g_ref[...], bb_ref[...], n)
        x_ref[...] = x
        y_ref[...] = jnp.dot(x, w_ref[...], preferred_element_type=_F32) + b_ref[...]

    return pl.pallas_call(
        body,
        grid=(_EGRID,),
        in_specs=[_row_spec(_EBLK, 128), _full_spec((8, 128)),
                  _full_spec((1, 128)), _full_spec((1, 128)),
                  _full_spec((128, 128)), _full_spec((1, 128))],
        out_specs=[_row_spec(_EBLK, 128), _row_spec(_EBLK, 128)],
        out_shape=[jax.ShapeDtypeStruct((_EPAD, 128), _F32),
                   jax.ShapeDtypeStruct((_EPAD, 128), _F32)],
    )(t, st, bnp["g"].reshape(1, -1), bnp["b"].reshape(1, -1),
      w, b.reshape(1, -1))


def _mlp(hs, hd, m0w, m0b, m1w, m1b, m2w, m2b):
    def body(hs_ref, hd_ref, w0_ref, b0_ref, w1_ref, b1_ref, w2_ref, b2_ref,
             o_ref):
        x = jnp.concatenate([hs_ref[...], hd_ref[...]], axis=1)
        x = jnp.maximum(jnp.dot(x, w0_ref[...], preferred_element_type=_F32)
                        + b0_ref[...], 0.0)
        x = jnp.maximum(jnp.dot(x, w1_ref[...], preferred_element_type=_F32)
                        + b1_ref[...], 0.0)
        o_ref[...] = jnp.dot(x, w2_ref[...], preferred_element_type=_F32) + b2_ref[...]

    return pl.pallas_call(
        body,
        grid=(_EGRID,),
        in_specs=[_row_spec(_EBLK, 128), _row_spec(_EBLK, 128),
                  _full_spec((256, 128)), _full_spec((1, 128)),
                  _full_spec((128, 64)), _full_spec((1, 64)),
                  _full_spec((64, 2)), _full_spec((1, 2))],
        out_specs=_row_spec(_EBLK, 2),
        out_shape=jax.ShapeDtypeStruct((_EPAD, 2), _F32),
    )(hs, hd, m0w, m0b.reshape(1, -1), m1w, m1b.reshape(1, -1),
      m2w, m2b.reshape(1, -1))


def _sc_gather2(tab_a, idx_a, tab_b, idx_b):
    wa = tab_a.shape[1]
    wb = tab_b.shape[1]
    mesh = plsc.VectorSubcoreMesh(core_axis_name="c", subcore_axis_name="s", num_cores=_NC, num_subcores=_NS)

    @functools.partial(
        pl.kernel, mesh=mesh,
        out_type=[jax.ShapeDtypeStruct((_EPAD, wa), _F32),
                  jax.ShapeDtypeStruct((_EPAD, wb), _F32)],
        scratch_types=[pltpu.VMEM((_NCHUNK, _CH), jnp.int32),
                       pltpu.VMEM((_NCHUNK, _CH), jnp.int32),
                       pltpu.VMEM((_CH, wa), _F32),
                       pltpu.VMEM((_CH, wa), _F32),
                       pltpu.VMEM((_CH, wb), _F32),
                       pltpu.VMEM((_CH, wb), _F32),
                       pltpu.SemaphoreType.DMA, pltpu.SemaphoreType.DMA,
                       pltpu.SemaphoreType.DMA, pltpu.SemaphoreType.DMA,
                       pltpu.SemaphoreType.DMA, pltpu.SemaphoreType.DMA,
                       pltpu.SemaphoreType.DMA, pltpu.SemaphoreType.DMA],
    )
    def run(ta, ia, tb, ib, oa, ob, ia_v, ib_v, a0, a1, b0, b1,
            gsa0, gsa1, gsb0, gsb1, wsa0, wsa1, wsb0, wsb1):
        wid = lax.axis_index("s") * _NC + lax.axis_index("c")
        pltpu.sync_copy(ia.at[wid], ia_v)
        pltpu.sync_copy(ib.at[wid], ib_v)
        base = wid * _EPT
        abufs, bbufs = (a0, a1), (b0, b1)
        gsa, gsb = (gsa0, gsa1), (gsb0, gsb1)
        wsa, wsb = (wsa0, wsa1), (wsb0, wsb1)

        def body(i, carry):
            descs = []
            for p in (0, 1):
                j = 2 * i + p

                @pl.when(i > 0)
                def _(p=p):
                    pltpu.make_async_copy(abufs[p], oa.at[pl.ds(0, _CH)],
                                          wsa[p]).wait()
                    pltpu.make_async_copy(bbufs[p], ob.at[pl.ds(0, _CH)],
                                          wsb[p]).wait()

                da = pltpu.async_copy(ta.at[ia_v.at[j]], abufs[p], gsa[p])
                db = pltpu.async_copy(tb.at[ib_v.at[j]], bbufs[p], gsb[p])
                descs.append((da, db))
            for p in (0, 1):
                off = base + (2 * i + p) * _CH
                da, db = descs[p]
                da.wait()
                pltpu.async_copy(abufs[p], oa.at[pl.ds(off, _CH)], wsa[p])
                db.wait()
                pltpu.async_copy(bbufs[p], ob.at[pl.ds(off, _CH)], wsb[p])
            return carry

        lax.fori_loop(0, _NCHUNK // 2, body, 0, unroll=False)
        for p in (0, 1):
            pltpu.make_async_copy(abufs[p], oa.at[pl.ds(0, _CH)], wsa[p]).wait()
            pltpu.make_async_copy(bbufs[p], ob.at[pl.ds(0, _CH)], wsb[p]).wait()

    return run(tab_a, idx_a, tab_b, idx_b)


def _sc_scatter(contrib, contribz, idx_s, idx_z, zeros128):
    mesh = plsc.VectorSubcoreMesh(core_axis_name="c", subcore_axis_name="s",
                                  num_cores=_NC, num_subcores=_NS)

    @functools.partial(
        pl.kernel, mesh=mesh,
        out_type=[jax.ShapeDtypeStruct((2, _NACC, 128), _F32),
                  jax.ShapeDtypeStruct((2, _ZACC, 128), _F32)],
        scratch_types=[pltpu.VMEM_SHARED((_NACC, 128), _F32),
                       pltpu.VMEM_SHARED((_ZACC, 128), _F32),
                       pltpu.VMEM((_NCHUNK, _CH), jnp.int32),
                       pltpu.VMEM((_NCHUNK, _CH), jnp.int32),
                       pltpu.VMEM((_CH, 128), _F32),
                       pltpu.VMEM((_CH, 128), _F32),
                       pltpu.SemaphoreType.DMA, pltpu.SemaphoreType.DMA],
    )
    def run(c_h, cz_h, idxs_h, idxz_h, z128, wv_out, z_out,
            accv, accz, idxs_v, idxz_v, cb, zb, lsc, lsz):
        cid = lax.axis_index("c")
        sid = lax.axis_index("s")
        wid = sid * _NC + cid

        for k in range(_NCH_ACC // _NS):
            coff = (sid + k * _NS) * _CH
            pltpu.sync_copy(z128, accv.at[pl.ds(coff, _CH)])

        @pl.when(sid < _ZACC // _CH)
        def _():
            pltpu.sync_copy(z128, accz.at[pl.ds(sid * _CH, _CH)])

        plsc.subcore_barrier()

        pltpu.sync_copy(idxs_h.at[wid], idxs_v)
        pltpu.sync_copy(idxz_h.at[wid], idxz_v)
        base = wid * _EPT
        pltpu.async_copy(c_h.at[pl.ds(base, _CH)], cb, lsc)

        def body(j, carry):
            off = base + j * _CH
            dz = pltpu.async_copy(cz_h.at[pl.ds(off, _CH)], zb, lsz)
            pltpu.make_async_copy(c_h.at[pl.ds(0, _CH)], cb, lsc).wait()
            pltpu.sync_copy(cb, accv.at[idxs_v.at[j]], add=True)

            @pl.when(j < _NCHUNK - 1)
            def _():
                pltpu.async_copy(c_h.at[pl.ds(off + _CH, _CH)], cb, lsc)

            dz.wait()
            pltpu.sync_copy(zb, accz.at[idxz_v.at[j]], add=True)
            return carry

        lax.fori_loop(0, _NCHUNK, body, 0, unroll=False)
        plsc.subcore_barrier()

        for k in range(_NCH_ACC // _NS):
            coff = (sid + k * _NS) * _CH
            pltpu.sync_copy(accv.at[pl.ds(coff, _CH)], wv_out.at[cid, pl.ds(coff, _CH)])

        @pl.when(sid < _ZACC // _CH)
        def _():
            pltpu.sync_copy(accz.at[pl.ds(sid * _CH, _CH)],
                            z_out.at[cid, pl.ds(sid * _CH, _CH)])

    return run(contrib, contribz, idx_s, idx_z, zeros128)


def kernel(edge_index, h, e, params):
    src = edge_index[0]
    dst = edge_index[1]
    pad = _EPAD - _E
    srcp = jnp.concatenate([src, jnp.zeros((pad,), jnp.int32)]).reshape(
        _NW, _NCHUNK, _CH)
    dstp = jnp.concatenate([dst, jnp.zeros((pad,), jnp.int32)]).reshape(
        _NW, _NCHUNK, _CH)
    dstf = jnp.concatenate([dst, jnp.full((pad,), _DUMMY, jnp.int32)])
    dsts = dstf.reshape(_NW, _NCHUNK, _CH)
    dstz = (dstf // 16).reshape(_NW, _NCHUNK, _CH)
    ohm = jax.nn.one_hot(dstf % 16, 16, dtype=_F32)
    ep = jnp.concatenate([e, jnp.zeros((pad, e.shape[1]), e.dtype)])
    zeros128 = jnp.zeros((_CH, 128), _F32)

    h = _linear(h, params["emb_h"]["w"], params["emb_h"]["b"], _NBLK)
    e = _linear(ep, params["emb_e"]["w"], params["emb_e"]["b"], _EBLK)

    layers = params["layers"]
    nl = len(layers)
    t2h = st2h = t2e = st2e = None
    for li, lp in enumerate(layers):
        last = li == nl - 1
        wkv = jnp.concatenate([lp["K"]["w"], lp["V"]["w"]], axis=1)
        bkv = jnp.concatenate([lp["K"]["b"], lp["V"]["b"]])
        if li == 0:
            hin, ein = h, e
            q, kv = _qkv(h, lp["Q"]["w"], lp["Q"]["b"], wkv, bkv)
            pe = _linear(e, lp["Epj"]["w"], lp["Epj"]["b"], _EBLK)
        else:
            prev = layers[li - 1]
            hin, q, kv = _bn_qkv(t2h, st2h, prev["bn2h"],
                                 lp["Q"]["w"], lp["Q"]["b"], wkv, bkv)
            ein, pe = _bn_linear2(t2e, st2e, prev["bn2e"],
                                  lp["Epj"]["w"], lp["Epj"]["b"], float(_E))
        qd, ksvs = _sc_gather2(q, dstp, kv, srcp)
        if last:
            c, cz = _score(qd, ksvs, pe, ohm, want_eatt=False)
        else:
            e_att, c, cz = _score(qd, ksvs, pe, ohm, want_eatt=True)
        wvp, zp = _sc_scatter(c, cz, dsts, dstz, zeros128)
        zp = zp.reshape(2, _NACC, 8)
        t1h, st1h = _hatt(wvp, zp, hin, lp["Oh"]["w"], lp["Oh"]["b"])
        t2h, st2h = _bn_ffn(t1h, st1h, lp["bn1h"],
                            lp["ffn_h1"]["w"], lp["ffn_h1"]["b"],
                            lp["ffn_h2"]["w"], lp["ffn_h2"]["b"],
                            float(_N), _N, _HBLK, _HGRID)
        if not last:
            t1e, st1e = _resid_linear(ein, e_att, lp["Oe"]["w"], lp["Oe"]["b"],
                                      _EPAD, _EBLK, _EREAL)
            t2e, st2e = _bn_ffn(t1e, st1e, lp["bn1e"],
                                lp["ffn_e1"]["w"], lp["ffn_e1"]["b"],
                                lp["ffn_e2"]["w"], lp["ffn_e2"]["b"],
                                float(_E), _EPAD, _EBLK, _EREAL)

    h = _bn_only(t2h, st2h, layers[-1]["bn2h"], float(_N), _N, _NBLK)
    hs, hd = _sc_gather2(h, srcp, h, dstp)
    mlp = params["mlp"]
    x = _mlp(hs, hd, mlp[0]["w"], mlp[0]["b"], mlp[1]["w"], mlp[1]["b"],
             mlp[2]["w"], mlp[2]["b"])
    return x[:_E]


# speedup vs baseline: 16.5434x; 1.0322x over previous
import functools

import jax
import jax.numpy as jnp
from jax import lax
from jax.experimental import pallas as pl
from jax.experimental.pallas import tpu as pltpu
from jax.experimental.pallas import tpu_sc as plsc

_N = 10000
_E = 160000
_HID = 128
_F32 = jnp.float32

_NC = 2
_NS = 16
_NW = _NC * _NS
_CH = 128
_NCHUNK = 40
_EPT = _NCHUNK * _CH
_EPAD = _NW * _EPT
_NACC = 10240
_NCH_ACC = _NACC // _CH
_ZACC = _NACC // 16
_DUMMY = _N

_EBLK = 1280
_EGRID = _EPAD // _EBLK
_EREAL = _E // _EBLK
_HBLK = 400
_HGRID = _N // _HBLK
_NBLK = 2000
_NGRID = _N // _NBLK


def _head_mat(rows, cols, div_axis):
    a = lax.broadcasted_iota(jnp.int32, (rows, cols), 0)
    b = lax.broadcasted_iota(jnp.int32, (rows, cols), 1)
    if div_axis == 1:
        b = b // 16
    else:
        a = a // 16
    return (a == b).astype(_F32)


def _expand_mats():
    bt = _head_mat(128, 8, 0)
    b8 = _head_mat(8, 128, 1)
    l8 = lax.broadcasted_iota(jnp.int32, (8, 128), 1)
    r8 = lax.broadcasted_iota(jnp.int32, (8, 128), 0)
    p1 = (l8 % 8 == r8).astype(_F32)
    l16 = lax.broadcasted_iota(jnp.int32, (16, 128), 1)
    r16 = lax.broadcasted_iota(jnp.int32, (16, 128), 0)
    p2 = (l16 // 8 == r16).astype(_F32)
    return bt, b8, p1, p2


def _row_spec(blk, width):
    return pl.BlockSpec((blk, width), lambda i: (i, 0))


def _full_spec(shape):
    return pl.BlockSpec(shape, lambda i: tuple(0 for _ in shape))


def _stats_update(st_ref, x, i, nreal_blocks):
    @pl.when(i == 0)
    def _():
        st_ref[...] = jnp.zeros_like(st_ref)

    @pl.when(i < nreal_blocks)
    def _():
        s = jnp.sum(x, axis=0, keepdims=True)
        s2 = jnp.sum(x * x, axis=0, keepdims=True)
        st_ref[0:1, :] += s
        st_ref[1:2, :] += s2


def _bn_apply(x, st, g, b, n):
    mean = st[0:1, :] * (1.0 / n)
    var = st[1:2, :] * (1.0 / n) - mean * mean
    return (x - mean) * lax.rsqrt(var + 1e-5) * g + b


def _linear(x, w, b, blk):
    rows, fin = x.shape
    fout = w.shape[1]

    def body(x_ref, w_ref, b_ref, o_ref):
        o_ref[...] = jnp.dot(x_ref[...], w_ref[...],
                             preferred_element_type=_F32) + b_ref[...]

    return pl.pallas_call(
        body,
        grid=(rows // blk,),
        in_specs=[_row_spec(blk, fin), _full_spec(w.shape), _full_spec((1, fout))],
        out_specs=_row_spec(blk, fout),
        out_shape=jax.ShapeDtypeStruct((rows, fout), _F32),
    )(x, w, b.reshape(1, -1))


def _qkv(h, lp):
    def body(x_ref, wq_ref, bq_ref, wk_ref, bk_ref, wv_ref, bv_ref,
             q_ref, k_ref, v_ref):
        x = x_ref[...]
        q_ref[...] = jnp.dot(x, wq_ref[...], preferred_element_type=_F32) + bq_ref[...]
        k_ref[...] = jnp.dot(x, wk_ref[...], preferred_element_type=_F32) + bk_ref[...]
        v_ref[...] = jnp.dot(x, wv_ref[...], preferred_element_type=_F32) + bv_ref[...]

    return pl.pallas_call(
        body,
        grid=(_NGRID,),
        in_specs=[_row_spec(_NBLK, 128)] + [_full_spec((128, 128)),
                  _full_spec((1, 128))] * 3,
        out_specs=[_row_spec(_NBLK, 128)] * 3,
        out_shape=[jax.ShapeDtypeStruct((_N, 128), _F32)] * 3,
    )(h, lp["Q"]["w"], lp["Q"]["b"].reshape(1, -1),
      lp["K"]["w"], lp["K"]["b"].reshape(1, -1),
      lp["V"]["w"], lp["V"]["b"].reshape(1, -1))


def _score(qd, ks, vs, pe, ohm, want_eatt):
    def body(qd_ref, ks_ref, vs_ref, pe_ref, ohm_ref, *outs):
        if want_eatt:
            eatt_ref, c_ref, cz_ref = outs
        else:
            c_ref, cz_ref = outs
        bt, b8, p1, p2 = _expand_mats()
        ks = ks_ref[...]
        vs = vs_ref[...]
        score = qd_ref[...] * ks * 0.25 * pe_ref[...]
        if want_eatt:
            eatt_ref[...] = score
        srow = jnp.clip(jnp.dot(score, bt, preferred_element_type=_F32),
                        -5.0, 5.0)
        sc = jnp.exp(srow)
        scex = jnp.dot(sc, b8, preferred_element_type=_F32)
        c_ref[...] = scex * vs
        cz_ref[...] = (jnp.dot(sc, p1, preferred_element_type=_F32)
                       * jnp.dot(ohm_ref[...], p2, preferred_element_type=_F32))

    out_shape = [jax.ShapeDtypeStruct((_EPAD, 128), _F32),
                 jax.ShapeDtypeStruct((_EPAD, 128), _F32)]
    out_specs = [_row_spec(_EBLK, 128), _row_spec(_EBLK, 128)]
    if want_eatt:
        out_shape = [jax.ShapeDtypeStruct((_EPAD, 128), _F32)] + out_shape
        out_specs = [_row_spec(_EBLK, 128)] + out_specs
    return pl.pallas_call(
        body,
        grid=(_EGRID,),
        in_specs=[_row_spec(_EBLK, 128), _row_spec(_EBLK, 128),
                  _row_spec(_EBLK, 128), _row_spec(_EBLK, 128),
                  _row_spec(_EBLK, 16)],
        out_specs=out_specs,
        out_shape=out_shape,
    )(qd, ks, vs, pe, ohm)


def _hatt(wvp, zp, hin, wo, bo):
    def body(wv_ref, z_ref, h_ref, wo_ref, bo_ref, t_ref, st_ref):
        i = pl.program_id(0)
        _, b8, _, _ = _expand_mats()
        wv = wv_ref[0] + wv_ref[1]
        z8 = z_ref[0] + z_ref[1]
        zex = jnp.dot(z8, b8, preferred_element_type=_F32)
        h_att = wv / (zex + 1e-6)
        t = h_ref[...] + jnp.dot(h_att, wo_ref[...],
                                 preferred_element_type=_F32) + bo_ref[...]
        t_ref[...] = t
        _stats_update(st_ref, t, i, _HGRID)

    return pl.pallas_call(
        body,
        grid=(_HGRID,),
        in_specs=[pl.BlockSpec((2, _HBLK, 128), lambda i: (0, i, 0)),
                  pl.BlockSpec((2, _HBLK, 8), lambda i: (0, i, 0)),
                  _row_spec(_HBLK, 128), _full_spec((128, 128)), _full_spec((1, 128))],
        out_specs=[_row_spec(_HBLK, 128), _full_spec((8, 128))],
        out_shape=[jax.ShapeDtypeStruct((_N, 128), _F32),
                   jax.ShapeDtypeStruct((8, 128), _F32)],
    )(wvp, zp, hin, wo, bo.reshape(1, -1))


def _resid_linear(xin, att, wo, bo, rows, blk, nreal_blocks):
    def body(x_ref, a_ref, wo_ref, bo_ref, t_ref, st_ref):
        i = pl.program_id(0)
        t = x_ref[...] + jnp.dot(a_ref[...], wo_ref[...],
                                 preferred_element_type=_F32) + bo_ref[...]
        t_ref[...] = t
        _stats_update(st_ref, t, i, nreal_blocks)

    return pl.pallas_call(
        body,
        grid=(rows // blk,),
        in_specs=[_row_spec(blk, 128), _row_spec(blk, 128),
                  _full_spec((128, 128)), _full_spec((1, 128))],
        out_specs=[_row_spec(blk, 128), _full_spec((8, 128))],
        out_shape=[jax.ShapeDtypeStruct((rows, 128), _F32),
                   jax.ShapeDtypeStruct((8, 128), _F32)],
    )(xin, att, wo, bo.reshape(1, -1))


def _bn_ffn(t, st, bnp, w1, b1, w2, b2, n, rows, blk, nreal_blocks):
    def body(t_ref, st_ref, g_ref, bb_ref, w1_ref, b1_ref, w2_ref, b2_ref,
             t2_ref, st2_ref):
        i = pl.program_id(0)
        x = _bn_apply(t_ref[...], st_ref[...], g_ref[...], bb_ref[...], n)
        u = jnp.maximum(jnp.dot(x, w1_ref[...], preferred_element_type=_F32)
                        + b1_ref[...], 0.0)
        t2 = x + jnp.dot(u, w2_ref[...], preferred_element_type=_F32) + b2_ref[...]
        t2_ref[...] = t2
        _stats_update(st2_ref, t2, i, nreal_blocks)

    return pl.pallas_call(
        body,
        grid=(rows // blk,),
        in_specs=[_row_spec(blk, 128), _full_spec((8, 128)),
                  _full_spec((1, 128)), _full_spec((1, 128)),
                  _full_spec((128, 256)), _full_spec((1, 256)),
                  _full_spec((256, 128)), _full_spec((1, 128))],
        out_specs=[_row_spec(blk, 128), _full_spec((8, 128))],
        out_shape=[jax.ShapeDtypeStruct((rows, 128), _F32),
                   jax.ShapeDtypeStruct((8, 128), _F32)],
    )(t, st, bnp["g"].reshape(1, -1), bnp["b"].reshape(1, -1),
      w1, b1.reshape(1, -1), w2, b2.reshape(1, -1))


def _bn_only(t, st, bnp, n, rows, blk):
    def body(t_ref, st_ref, g_ref, bb_ref, o_ref):
        o_ref[...] = _bn_apply(t_ref[...], st_ref[...], g_ref[...], bb_ref[...], n)

    return pl.pallas_call(
        body,
        grid=(rows // blk,),
        in_specs=[_row_spec(blk, 128), _full_spec((8, 128)),
                  _full_spec((1, 128)), _full_spec((1, 128))],
        out_specs=_row_spec(blk, 128),
        out_shape=jax.ShapeDtypeStruct((rows, 128), _F32),
    )(t, st, bnp["g"].reshape(1, -1), bnp["b"].reshape(1, -1))


def _bn_qkv(t, st, bnp, lp):
    def body(t_ref, st_ref, g_ref, bb_ref, wq_ref, bq_ref, wk_ref, bk_ref,
             wv_ref, bv_ref, h_ref, q_ref, k_ref, v_ref):
        x = _bn_apply(t_ref[...], st_ref[...], g_ref[...], bb_ref[...],
                      float(_N))
        h_ref[...] = x
        q_ref[...] = jnp.dot(x, wq_ref[...], preferred_element_type=_F32) + bq_ref[...]
        k_ref[...] = jnp.dot(x, wk_ref[...], preferred_element_type=_F32) + bk_ref[...]
        v_ref[...] = jnp.dot(x, wv_ref[...], preferred_element_type=_F32) + bv_ref[...]

    return pl.pallas_call(
        body,
        grid=(_NGRID,),
        in_specs=[_row_spec(_NBLK, 128), _full_spec((8, 128)),
                  _full_spec((1, 128)), _full_spec((1, 128))]
                 + [_full_spec((128, 128)), _full_spec((1, 128))] * 3,
        out_specs=[_row_spec(_NBLK, 128)] * 4,
        out_shape=[jax.ShapeDtypeStruct((_N, 128), _F32)] * 4,
    )(t, st, bnp["g"].reshape(1, -1), bnp["b"].reshape(1, -1),
      lp["Q"]["w"], lp["Q"]["b"].reshape(1, -1),
      lp["K"]["w"], lp["K"]["b"].reshape(1, -1),
      lp["V"]["w"], lp["V"]["b"].reshape(1, -1))


def _bn_linear2(t, st, bnp, w, b, n):
    def body(t_ref, st_ref, g_ref, bb_ref, w_ref, b_ref, x_ref, y_ref):
        x = _bn_apply(t_ref[...], st_ref[...], g_ref[...], bb_ref[...], n)
        x_ref[...] = x
        y_ref[...] = jnp.dot(x, w_ref[...], preferred_element_type=_F32) + b_ref[...]

    return pl.pallas_call(
        body,
        grid=(_EGRID,),
        in_specs=[_row_spec(_EBLK, 128), _full_spec((8, 128)),
                  _full_spec((1, 128)), _full_spec((1, 128)),
                  _full_spec((128, 128)), _full_spec((1, 128))],
        out_specs=[_row_spec(_EBLK, 128), _row_spec(_EBLK, 128)],
        out_shape=[jax.ShapeDtypeStruct((_EPAD, 128), _F32),
                   jax.ShapeDtypeStruct((_EPAD, 128), _F32)],
    )(t, st, bnp["g"].reshape(1, -1), bnp["b"].reshape(1, -1),
      w, b.reshape(1, -1))


def _mlp(hs, hd, m0w, m0b, m1w, m1b, m2w, m2b):
    def body(hs_ref, hd_ref, w0_ref, b0_ref, w1_ref, b1_ref, w2_ref, b2_ref,
             o_ref):
        x = jnp.concatenate([hs_ref[...], hd_ref[...]], axis=1)
        x = jnp.maximum(jnp.dot(x, w0_ref[...], preferred_element_type=_F32)
                        + b0_ref[...], 0.0)
        x = jnp.maximum(jnp.dot(x, w1_ref[...], preferred_element_type=_F32)
                        + b1_ref[...], 0.0)
        o_ref[...] = jnp.dot(x, w2_ref[...], preferred_element_type=_F32) + b2_ref[...]

    return pl.pallas_call(
        body,
        grid=(_EGRID,),
        in_specs=[_row_spec(_EBLK, 128), _row_spec(_EBLK, 128),
                  _full_spec((256, 128)), _full_spec((1, 128)),
                  _full_spec((128, 64)), _full_spec((1, 64)),
                  _full_spec((64, 2)), _full_spec((1, 2))],
        out_specs=_row_spec(_EBLK, 2),
        out_shape=jax.ShapeDtypeStruct((_EPAD, 2), _F32),
    )(hs, hd, m0w, m0b.reshape(1, -1), m1w, m1b.reshape(1, -1),
      m2w, m2b.reshape(1, -1))


def _sc_gather_multi(tabs, idxs):
    nt = len(tabs)
    widths = [t.shape[1] for t in tabs]
    mesh = plsc.VectorSubcoreMesh(core_axis_name="c", subcore_axis_name="s",
                                  num_cores=_NC, num_subcores=_NS)
    scratch = ([pltpu.VMEM((_NCHUNK, _CH), jnp.int32)] * nt
               + [pltpu.VMEM((_CH, w), _F32) for w in widths for _ in (0, 1)]
               + [pltpu.SemaphoreType.DMA] * (4 * nt))

    @functools.partial(
        pl.kernel, mesh=mesh,
        out_type=[jax.ShapeDtypeStruct((_EPAD, w), _F32) for w in widths],
        scratch_types=scratch,
    )
    def run(*args):
        tabs_r = args[:nt]
        idxs_r = args[nt:2 * nt]
        outs = args[2 * nt:3 * nt]
        sc = args[3 * nt:]
        idx_v = sc[:nt]
        bufs = sc[nt:3 * nt]
        gs = sc[3 * nt:5 * nt]
        ws = sc[5 * nt:7 * nt]
        wid = lax.axis_index("s") * _NC + lax.axis_index("c")
        for t in range(nt):
            pltpu.sync_copy(idxs_r[t].at[wid], idx_v[t])
        base = wid * _EPT

        def body(i, carry):
            descs = []
            for p in (0, 1):
                j = 2 * i + p

                @pl.when(i > 0)
                def _(p=p):
                    for t in range(nt):
                        pltpu.make_async_copy(bufs[2 * t + p],
                                              outs[t].at[pl.ds(0, _CH)],
                                              ws[2 * t + p]).wait()

                for t in range(nt):
                    descs.append(pltpu.async_copy(
                        tabs_r[t].at[idx_v[t].at[j]], bufs[2 * t + p],
                        gs[2 * t + p]))
            for p in (0, 1):
                off = base + (2 * i + p) * _CH
                for t in range(nt):
                    descs[p * nt + t].wait()
                    pltpu.async_copy(bufs[2 * t + p],
                                     outs[t].at[pl.ds(off, _CH)], ws[2 * t + p])
            return carry

        lax.fori_loop(0, _NCHUNK // 2, body, 0, unroll=False)
        for p in (0, 1):
            for t in range(nt):
                pltpu.make_async_copy(bufs[2 * t + p],
                                      outs[t].at[pl.ds(0, _CH)],
                                      ws[2 * t + p]).wait()

    return run(*tabs, *idxs)


def _sc_scatter(contrib, contribz, idx_s, idx_z, zeros128):
    mesh = plsc.VectorSubcoreMesh(core_axis_name="c", subcore_axis_name="s",
                                  num_cores=_NC, num_subcores=_NS)

    @functools.partial(
        pl.kernel, mesh=mesh,
        out_type=[jax.ShapeDtypeStruct((2, _NACC, 128), _F32),
                  jax.ShapeDtypeStruct((2, _ZACC, 128), _F32)],
        scratch_types=[pltpu.VMEM_SHARED((_NACC, 128), _F32),
                       pltpu.VMEM_SHARED((_ZACC, 128), _F32),
                       pltpu.VMEM((_NCHUNK, _CH), jnp.int32),
                       pltpu.VMEM((_NCHUNK, _CH), jnp.int32),
                       pltpu.VMEM((_CH, 128), _F32),
                       pltpu.VMEM((_CH, 128), _F32),
                       pltpu.SemaphoreType.DMA, pltpu.SemaphoreType.DMA],
    )
    def run(c_h, cz_h, idxs_h, idxz_h, z128, wv_out, z_out,
            accv, accz, idxs_v, idxz_v, cb, zb, lsc, lsz):
        cid = lax.axis_index("c")
        sid = lax.axis_index("s")
        wid = sid * _NC + cid

        for k in range(_NCH_ACC // _NS):
            coff = (sid + k * _NS) * _CH
            pltpu.sync_copy(z128, accv.at[pl.ds(coff, _CH)])

        @pl.when(sid < _ZACC // _CH)
        def _():
            pltpu.sync_copy(z128, accz.at[pl.ds(sid * _CH, _CH)])

        plsc.subcore_barrier()

        pltpu.sync_copy(idxs_h.at[wid], idxs_v)
        pltpu.sync_copy(idxz_h.at[wid], idxz_v)
        base = wid * _EPT
        pltpu.async_copy(c_h.at[pl.ds(base, _CH)], cb, lsc)

        def body(j, carry):
            off = base + j * _CH
            dz = pltpu.async_copy(cz_h.at[pl.ds(off, _CH)], zb, lsz)
            pltpu.make_async_copy(c_h.at[pl.ds(0, _CH)], cb, lsc).wait()
            pltpu.sync_copy(cb, accv.at[idxs_v.at[j]], add=True)

            @pl.when(j < _NCHUNK - 1)
            def _():
                pltpu.async_copy(c_h.at[pl.ds(off + _CH, _CH)], cb, lsc)

            dz.wait()
            pltpu.sync_copy(zb, accz.at[idxz_v.at[j]], add=True)
            return carry

        lax.fori_loop(0, _NCHUNK, body, 0, unroll=False)
        plsc.subcore_barrier()

        for k in range(_NCH_ACC // _NS):
            coff = (sid + k * _NS) * _CH
            pltpu.sync_copy(accv.at[pl.ds(coff, _CH)], wv_out.at[cid, pl.ds(coff, _CH)])

        @pl.when(sid < _ZACC // _CH)
        def _():
            pltpu.sync_copy(accz.at[pl.ds(sid * _CH, _CH)],
                            z_out.at[cid, pl.ds(sid * _CH, _CH)])

    return run(contrib, contribz, idx_s, idx_z, zeros128)


def kernel(edge_index, h, e, params):
    src = edge_index[0]
    dst = edge_index[1]
    pad = _EPAD - _E
    srcp = jnp.concatenate([src, jnp.zeros((pad,), jnp.int32)]).reshape(
        _NW, _NCHUNK, _CH)
    dstp = jnp.concatenate([dst, jnp.zeros((pad,), jnp.int32)]).reshape(
        _NW, _NCHUNK, _CH)
    dstf = jnp.concatenate([dst, jnp.full((pad,), _DUMMY, jnp.int32)])
    dsts = dstf.reshape(_NW, _NCHUNK, _CH)
    dstz = (dstf // 16).reshape(_NW, _NCHUNK, _CH)
    ohm = jax.nn.one_hot(dstf % 16, 16, dtype=_F32)
    ep = jnp.concatenate([e, jnp.zeros((pad, e.shape[1]), e.dtype)])
    zeros128 = jnp.zeros((_CH, 128), _F32)

    h = _linear(h, params["emb_h"]["w"], params["emb_h"]["b"], _NBLK)
    e = _linear(ep, params["emb_e"]["w"], params["emb_e"]["b"], _EBLK)

    layers = params["layers"]
    nl = len(layers)
    t2h = st2h = t2e = st2e = None
    for li, lp in enumerate(layers):
        last = li == nl - 1
        if li == 0:
            hin, ein = h, e
            q, k, v = _qkv(h, lp)
            pe = _linear(e, lp["Epj"]["w"], lp["Epj"]["b"], _EBLK)
        else:
            prev = layers[li - 1]
            hin, q, k, v = _bn_qkv(t2h, st2h, prev["bn2h"], lp)
            ein, pe = _bn_linear2(t2e, st2e, prev["bn2e"],
                                  lp["Epj"]["w"], lp["Epj"]["b"], float(_E))
        qd, ks, vs = _sc_gather_multi([q, k, v], [dstp, srcp, srcp])
        if last:
            c, cz = _score(qd, ks, vs, pe, ohm, want_eatt=False)
        else:
            e_att, c, cz = _score(qd, ks, vs, pe, ohm, want_eatt=True)
        wvp, zp = _sc_scatter(c, cz, dsts, dstz, zeros128)
        zp = zp.reshape(2, _NACC, 8)
        t1h, st1h = _hatt(wvp, zp, hin, lp["Oh"]["w"], lp["Oh"]["b"])
        t2h, st2h = _bn_ffn(t1h, st1h, lp["bn1h"],
                            lp["ffn_h1"]["w"], lp["ffn_h1"]["b"],
                            lp["ffn_h2"]["w"], lp["ffn_h2"]["b"],
                            float(_N), _N, _HBLK, _HGRID)
        if not last:
            t1e, st1e = _resid_linear(ein, e_att, lp["Oe"]["w"], lp["Oe"]["b"],
                                      _EPAD, _EBLK, _EREAL)
            t2e, st2e = _bn_ffn(t1e, st1e, lp["bn1e"],
                                lp["ffn_e1"]["w"], lp["ffn_e1"]["b"],
                                lp["ffn_e2"]["w"], lp["ffn_e2"]["b"],
                                float(_E), _EPAD, _EBLK, _EREAL)

    h = _bn_only(t2h, st2h, layers[-1]["bn2h"], float(_N), _N, _NBLK)
    hs, hd = _sc_gather_multi([h, h], [srcp, dstp])
    mlp = params["mlp"]
    x = _mlp(hs, hd, mlp[0]["w"], mlp[0]["b"], mlp[1]["w"], mlp[1]["b"],
             mlp[2]["w"], mlp[2]["b"])
    return x[:_E]
